# Initial kernel scaffold; baseline (speedup 1.0000x reference)
import jax
import jax.numpy as jnp
from jax.experimental import pallas as pl


def kernel(x, W_qkv, gq, bq, gk, bk, gv, bv, Wp, bp, gp, bpn):
    raise NotImplementedError("write your pallas kernel here")


# speedup vs baseline: 7.9819x; 7.9819x over previous
import functools

import jax
import jax.numpy as jnp
from jax.experimental import pallas as pl
from jax.experimental.pallas import tpu as pltpu

DIM = 256
HEADS = 8
HEAD_DIM = DIM // HEADS
NWIN = (2, 4, 4)
TOPK_WIN = 4
VTH = 0.5
TAU = 2.0
SCALE = HEAD_DIM ** (-0.5)

W = NWIN[0] * NWIN[1] * NWIN[2]
_INTERPRET = False

_LT, _LH, _LW = 8, 16, 16
_GT, _GH, _GW = _LT // NWIN[0], _LH // NWIN[1], _LW // NWIN[2]


def _fwd_piece_starts():
    starts = []
    for wt in range(NWIN[0]):
        for wh in range(NWIN[1]):
            for ww in range(NWIN[2]):
                for it in range(_GT):
                    for ih in range(_GH):
                        starts.append(((wt * _GT + it) * _LH
                                       + (wh * _GH + ih)) * _LW + ww * _GW)
    return starts


def _inv_piece_starts():
    ws = _GT * _GH * _GW
    starts = []
    for lt in range(_LT):
        for lh in range(_LH):
            for ww in range(NWIN[2]):
                w = (lt // _GT) * NWIN[1] * NWIN[2] + (lh // _GH) * NWIN[2] + ww
                starts.append(w * ws + (lt % _GT) * _GH * _GW
                              + (lh % _GH) * _GW)
    return starts


_FWD = _fwd_piece_starts()
_INV = _inv_piece_starts()


def _permute_rows(val, starts, run):
    return jnp.concatenate([val[s:s + run, :] for s in starts], axis=0)


def _win_id_iota(shape, dim):
    s = jax.lax.broadcasted_iota(jnp.int32, shape, dim)
    lt = s // (_LH * _LW)
    lh = (s // _LW) % _LH
    lw = s % _LW
    return ((lt // _GT) * NWIN[1] * NWIN[2]
            + (lh // _GH) * NWIN[2] + (lw // _GW))


def _ln(xv, g, b, eps=1e-5):
    m = jnp.mean(xv, axis=-1, keepdims=True)
    c = xv - m
    v = jnp.mean(c * c, axis=-1, keepdims=True)
    return c * jax.lax.rsqrt(v + eps) * g + b


def _spike_ln(xv, g, b, dtype, eps=1e-5):
    m = jnp.mean(xv, axis=-1, keepdims=True)
    c = xv - m
    v = jnp.mean(c * c, axis=-1, keepdims=True)
    thr = (VTH * TAU - b) * jnp.sqrt(v + eps)
    return jnp.where(c * g >= thr, 1.0, 0.0).astype(dtype)


def _routing_kernel(x_ref, idx_ref, r_ref, *, tb, b_count, ws):
    n_rows = W * ws
    t_count = tb // b_count
    sums = [None] * b_count
    for i in range(tb):
        chunk = x_ref[i * n_rows:(i + 1) * n_rows, :]
        b = i % b_count
        sums[b] = chunk if sums[b] is None else sums[b] + chunk
    for b in range(b_count):
        sums[b] = _permute_rows(sums[b], _FWD, _GW)
    fold = ws
    while fold > 8:
        fold //= 2
        for b in range(b_count):
            a = sums[b]
            sums[b] = jnp.concatenate(
                [a[w * 2 * fold:w * 2 * fold + fold, :]
                 + a[w * 2 * fold + fold:(w + 1) * 2 * fold, :]
                 for w in range(W)], axis=0)
    srow = jax.lax.broadcasted_iota(jnp.int32, (W, W * fold), 1)
    wrow = jax.lax.broadcasted_iota(jnp.int32, (W, W * fold), 0)
    S = jnp.where(srow // fold == wrow, 1.0, 0.0).astype(jnp.float32)
    colidx = jax.lax.broadcasted_iota(jnp.int32, (W, W), 1)
    for b in range(b_count):
        region = jnp.dot(S, sums[b], preferred_element_type=jnp.float32,
                         precision=jax.lax.Precision.HIGHEST)
        region = region * (1.0 / (t_count * ws))
        region16 = region.astype(jnp.bfloat16)
        scores = jax.lax.dot_general(
            region16, region16, (((1,), (1,)), ((), ())),
            preferred_element_type=jnp.float32) * SCALE
        masked = scores
        Rb = jnp.zeros((W, W), jnp.float32)
        for j in range(TOPK_WIN):
            mval = jnp.max(masked, axis=1, keepdims=True)
            eq = masked == mval
            cand = jnp.where(eq, colidx, jnp.int32(10 ** 9))
            cmin = jnp.min(cand, axis=1, keepdims=True)
            onehot = colidx == cmin
            Rb = Rb + onehot.astype(jnp.float32)
            masked = jnp.where(onehot, -jnp.inf, masked)
            idx_ref[b, :, j:j + 1] = cmin
        r_ref[b] = Rb


def _main_kernel(idx_sref, x_ref, wqkv_ref, gq_ref, bq_ref, gk_ref, bk_ref,
                 gv_ref, bv_ref, r_ref, wp_ref, bp_ref, gp_ref, bpn_ref,
                 y_ref, kvb_ref, out_ref, *, ws, b_count):
    b = pl.program_id(0) % b_count
    xv = _permute_rows(x_ref[...], _FWD, _GW)
    qkv = jnp.dot(xv, wqkv_ref[...],
                  preferred_element_type=jnp.float32)
    bf = jnp.bfloat16
    q16 = _spike_ln(qkv[:, 0:DIM], gq_ref[...], bq_ref[...], bf)
    k = _spike_ln(qkv[:, DIM:2 * DIM], gk_ref[...], bk_ref[...], bf)
    v = _spike_ln(qkv[:, 2 * DIM:3 * DIM], gv_ref[...], bv_ref[...], bf)

    n_rows = W * ws
    srow = jax.lax.broadcasted_iota(jnp.int32, (W, n_rows), 1)
    wrow = jax.lax.broadcasted_iota(jnp.int32, (W, n_rows), 0)
    S = jnp.where(srow // ws == wrow, 1.0, 0.0).astype(bf)
    ks = jnp.dot(S, k, preferred_element_type=jnp.float32)

    for w in range(W):
        kw = k[w * ws:(w + 1) * ws, :]
        vw = v[w * ws:(w + 1) * ws, :]
        kv_full = jax.lax.dot_general(
            kw, vw, (((0,), (0,)), ((), ())),
            preferred_element_type=jnp.float32)
        br = jnp.concatenate(
            [kv_full[h * HEAD_DIM:(h + 1) * HEAD_DIM,
                     h * HEAD_DIM:(h + 1) * HEAD_DIM]
             for h in range(HEADS)], axis=1)
        kvb_ref[w * HEAD_DIM:(w + 1) * HEAD_DIM, :] = br.astype(jnp.bfloat16)

    R = r_ref[0]
    ksg = jnp.dot(R, ks, preferred_element_type=jnp.float32)
    rrow = jax.lax.broadcasted_iota(jnp.int32, (n_rows, W), 0)
    rcol = jax.lax.broadcasted_iota(jnp.int32, (n_rows, W), 1)
    Urows = jnp.where(rrow // ws == rcol, 1.0, 0.0).astype(bf)
    ksg_exp = jnp.dot(Urows, ksg.astype(bf),
                      preferred_element_type=jnp.float32).astype(bf)
    ci = jax.lax.broadcasted_iota(jnp.int32, (DIM, HEADS), 0)
    hi = jax.lax.broadcasted_iota(jnp.int32, (DIM, HEADS), 1)
    Esum = jnp.where(ci // HEAD_DIM == hi, 1.0, 0.0).astype(jnp.float32)
    den8 = jnp.dot(q16 * ksg_exp, Esum.astype(bf),
                   preferred_element_type=jnp.float32)
    den = jnp.dot(den8, jnp.transpose(Esum),
                  preferred_element_type=jnp.float32)

    for w in range(W):
        j0 = idx_sref[b, w, 0]
        kvgw = kvb_ref[pl.ds(j0 * HEAD_DIM, HEAD_DIM), :]
        for j in range(1, TOPK_WIN):
            jj = idx_sref[b, w, j]
            kvgw = kvgw + kvb_ref[pl.ds(jj * HEAD_DIM, HEAD_DIM), :]
        qw = q16[w * ws:(w + 1) * ws, :]
        num = jnp.concatenate(
            [jnp.dot(qw[:, h * HEAD_DIM:(h + 1) * HEAD_DIM],
                     kvgw[:, h * HEAD_DIM:(h + 1) * HEAD_DIM],
                     preferred_element_type=jnp.float32)
             for h in range(HEADS)], axis=1)
        out_ref[w * ws:(w + 1) * ws, :] = num

    attn = (out_ref[...] * SCALE) / (jnp.abs(den * SCALE) + 1e-4)
    attn = _permute_rows(attn, _INV, _GW)
    y = jnp.dot(attn, wp_ref[...],
                preferred_element_type=jnp.float32) + bp_ref[...]
    y_ref[...] = _ln(y, gp_ref[...], bpn_ref[...])


@jax.jit
def kernel(x, W_qkv, gq, bq, gk, bk, gv, bv, Wp, bp, gp, bpn):
    T, B, Lt, Lh, Lw, C = x.shape
    wt, wh, ww = NWIN
    ws = (Lt // wt) * (Lh // wh) * (Lw // ww)
    TB = T * B
    n_rows = W * ws

    x2 = x.reshape(TB * n_rows, C)

    routing = pl.pallas_call(
        functools.partial(_routing_kernel, tb=TB, b_count=B, ws=ws),
        out_shape=(jax.ShapeDtypeStruct((B, W, TOPK_WIN), jnp.int32),
                   jax.ShapeDtypeStruct((B, W, W), jnp.float32)),
        in_specs=[pl.BlockSpec((TB * n_rows, C), lambda: (0, 0))],
        out_specs=(pl.BlockSpec((B, W, TOPK_WIN), lambda: (0, 0, 0)),
                   pl.BlockSpec((B, W, W), lambda: (0, 0, 0))),
        interpret=_INTERPRET,
    )
    idx, R = routing(x2)

    v1 = lambda a: a.reshape(1, C)
    full2 = lambda shape: pl.BlockSpec(shape, lambda i, idx_ref: (0, 0))

    main = pl.pallas_call(
        functools.partial(_main_kernel, ws=ws, b_count=B),
        grid_spec=pltpu.PrefetchScalarGridSpec(
            num_scalar_prefetch=1,
            grid=(TB,),
            in_specs=[
                pl.BlockSpec((n_rows, C), lambda i, idx_ref: (i, 0)),
                full2((C, 3 * C)),
                full2((1, C)), full2((1, C)),
                full2((1, C)), full2((1, C)),
                full2((1, C)), full2((1, C)),
                pl.BlockSpec((1, W, W),
                             lambda i, idx_ref: (i % B, 0, 0)),
                full2((C, C)),
                full2((1, C)), full2((1, C)), full2((1, C)),
            ],
            out_specs=pl.BlockSpec((n_rows, C), lambda i, idx_ref: (i, 0)),
            scratch_shapes=[
                pltpu.VMEM((W * HEAD_DIM, C), jnp.bfloat16),
                pltpu.VMEM((n_rows, C), jnp.float32),
            ],
        ),
        out_shape=jax.ShapeDtypeStruct((TB * n_rows, C), jnp.float32),
        compiler_params=pltpu.CompilerParams(
            dimension_semantics=("parallel",)),
        interpret=_INTERPRET,
    )
    y2 = main(idx, x2, W_qkv, v1(gq), v1(bq), v1(gk), v1(bk), v1(gv), v1(bv),
              R, Wp, v1(bp), v1(gp), v1(bpn))

    return y2.reshape(T, B, Lt, Lh, Lw, C)


# speedup vs baseline: 7.9875x; 1.0007x over previous
import functools

import jax
import jax.numpy as jnp
from jax.experimental import pallas as pl
from jax.experimental.pallas import tpu as pltpu

DIM = 256
HEADS = 8
HEAD_DIM = DIM // HEADS
NWIN = (2, 4, 4)
TOPK_WIN = 4
VTH = 0.5
TAU = 2.0
SCALE = HEAD_DIM ** (-0.5)

W = NWIN[0] * NWIN[1] * NWIN[2]

_LT, _LH, _LW = 8, 16, 16
_GT, _GH, _GW = _LT // NWIN[0], _LH // NWIN[1], _LW // NWIN[2]


def _fwd_piece_starts():
    starts = []
    for wt in range(NWIN[0]):
        for wh in range(NWIN[1]):
            for ww in range(NWIN[2]):
                for it in range(_GT):
                    for ih in range(_GH):
                        starts.append(((wt * _GT + it) * _LH
                                       + (wh * _GH + ih)) * _LW + ww * _GW)
    return starts


def _inv_piece_starts():
    ws = _GT * _GH * _GW
    starts = []
    for lt in range(_LT):
        for lh in range(_LH):
            for ww in range(NWIN[2]):
                w = (lt // _GT) * NWIN[1] * NWIN[2] + (lh // _GH) * NWIN[2] + ww
                starts.append(w * ws + (lt % _GT) * _GH * _GW
                              + (lh % _GH) * _GW)
    return starts


_FWD = _fwd_piece_starts()
_INV = _inv_piece_starts()


def _permute_rows(val, starts, run):
    return jnp.concatenate([val[s:s + run, :] for s in starts], axis=0)


def _win_id_iota(shape, dim):
    s = jax.lax.broadcasted_iota(jnp.int32, shape, dim)
    lt = s // (_LH * _LW)
    lh = (s // _LW) % _LH
    lw = s % _LW
    return ((lt // _GT) * NWIN[1] * NWIN[2]
            + (lh // _GH) * NWIN[2] + (lw // _GW))


def _ln(xv, g, b, eps=1e-5):
    m = jnp.mean(xv, axis=-1, keepdims=True)
    c = xv - m
    v = jnp.mean(c * c, axis=-1, keepdims=True)
    return c * jax.lax.rsqrt(v + eps) * g + b


def _spike_ln(xv, g, b, dtype, eps=1e-5):
    m = jnp.mean(xv, axis=-1, keepdims=True)
    c = xv - m
    v = jnp.mean(c * c, axis=-1, keepdims=True)
    thr = (VTH * TAU - b) * jnp.sqrt(v + eps)
    return jnp.where(c * g >= thr, 1.0, 0.0).astype(dtype)


def _routing_kernel(x_ref, idx_ref, r_ref, *, tb, b_count, ws):
    n_rows = W * ws
    t_count = tb // b_count
    sums = [None] * b_count
    for i in range(tb):
        chunk = x_ref[i * n_rows:(i + 1) * n_rows, :]
        b = i % b_count
        sums[b] = chunk if sums[b] is None else sums[b] + chunk
    for b in range(b_count):
        sums[b] = _permute_rows(sums[b], _FWD, _GW)
    fold = ws
    while fold > 8:
        fold //= 2
        for b in range(b_count):
            a = sums[b]
            sums[b] = jnp.concatenate(
                [a[w * 2 * fold:w * 2 * fold + fold, :]
                 + a[w * 2 * fold + fold:(w + 1) * 2 * fold, :]
                 for w in range(W)], axis=0)
    srow = jax.lax.broadcasted_iota(jnp.int32, (W, W * fold), 1)
    wrow = jax.lax.broadcasted_iota(jnp.int32, (W, W * fold), 0)
    S = jnp.where(srow // fold == wrow, 1.0, 0.0).astype(jnp.float32)
    colidx = jax.lax.broadcasted_iota(jnp.int32, (W, W), 1)
    for b in range(b_count):
        region = jnp.dot(S, sums[b], preferred_element_type=jnp.float32,
                         precision=jax.lax.Precision.HIGHEST)
        region = region * (1.0 / (t_count * ws))
        region16 = region.astype(jnp.bfloat16)
        scores = jax.lax.dot_general(
            region16, region16, (((1,), (1,)), ((), ())),
            preferred_element_type=jnp.float32) * SCALE
        masked = scores
        Rb = jnp.zeros((W, W), jnp.float32)
        for j in range(TOPK_WIN):
            mval = jnp.max(masked, axis=1, keepdims=True)
            eq = masked == mval
            cand = jnp.where(eq, colidx, jnp.int32(10 ** 9))
            cmin = jnp.min(cand, axis=1, keepdims=True)
            onehot = colidx == cmin
            Rb = Rb + onehot.astype(jnp.float32)
            masked = jnp.where(onehot, -jnp.inf, masked)
            idx_ref[b, :, j:j + 1] = cmin
        r_ref[b] = Rb


def _main_kernel(idx_sref, x_ref, wqkv_ref, gq_ref, bq_ref, gk_ref, bk_ref,
                 gv_ref, bv_ref, r_ref, wp_ref, bp_ref, gp_ref, bpn_ref,
                 y_ref, kvb_ref, out_ref, *, ws, b_count):
    b = pl.program_id(0) % b_count
    xv = _permute_rows(x_ref[...], _FWD, _GW)
    qkv = jnp.dot(xv, wqkv_ref[...],
                  preferred_element_type=jnp.float32)
    bf = jnp.bfloat16
    q16 = _spike_ln(qkv[:, 0:DIM], gq_ref[...], bq_ref[...], bf)
    k = _spike_ln(qkv[:, DIM:2 * DIM], gk_ref[...], bk_ref[...], bf)
    v = _spike_ln(qkv[:, 2 * DIM:3 * DIM], gv_ref[...], bv_ref[...], bf)

    n_rows = W * ws
    srow = jax.lax.broadcasted_iota(jnp.int32, (W, n_rows), 1)
    wrow = jax.lax.broadcasted_iota(jnp.int32, (W, n_rows), 0)
    S = jnp.where(srow // ws == wrow, 1.0, 0.0).astype(bf)
    ks = jnp.dot(S, k, preferred_element_type=jnp.float32)

    for w in range(W):
        kw = k[w * ws:(w + 1) * ws, :]
        vw = v[w * ws:(w + 1) * ws, :]
        kv_full = jax.lax.dot_general(
            kw, vw, (((0,), (0,)), ((), ())),
            preferred_element_type=jnp.float32)
        br = jnp.concatenate(
            [kv_full[h * HEAD_DIM:(h + 1) * HEAD_DIM,
                     h * HEAD_DIM:(h + 1) * HEAD_DIM]
             for h in range(HEADS)], axis=1)
        kvb_ref[w * HEAD_DIM:(w + 1) * HEAD_DIM, :] = br.astype(jnp.bfloat16)

    R = r_ref[0]
    ksg = jnp.dot(R, ks, preferred_element_type=jnp.float32)
    rrow = jax.lax.broadcasted_iota(jnp.int32, (n_rows, W), 0)
    rcol = jax.lax.broadcasted_iota(jnp.int32, (n_rows, W), 1)
    Urows = jnp.where(rrow // ws == rcol, 1.0, 0.0).astype(bf)
    ksg_exp = jnp.dot(Urows, ksg.astype(bf),
                      preferred_element_type=jnp.float32).astype(bf)
    ci = jax.lax.broadcasted_iota(jnp.int32, (DIM, HEADS), 0)
    hi = jax.lax.broadcasted_iota(jnp.int32, (DIM, HEADS), 1)
    Esum = jnp.where(ci // HEAD_DIM == hi, 1.0, 0.0).astype(jnp.float32)
    den8 = jnp.dot(q16 * ksg_exp, Esum.astype(bf),
                   preferred_element_type=jnp.float32)
    den = jnp.dot(den8, jnp.transpose(Esum),
                  preferred_element_type=jnp.float32)

    for w in range(W):
        j0 = idx_sref[b, w, 0]
        kvgw = kvb_ref[pl.ds(j0 * HEAD_DIM, HEAD_DIM), :]
        for j in range(1, TOPK_WIN):
            jj = idx_sref[b, w, j]
            kvgw = kvgw + kvb_ref[pl.ds(jj * HEAD_DIM, HEAD_DIM), :]
        qw = q16[w * ws:(w + 1) * ws, :]
        num = jnp.concatenate(
            [jnp.dot(qw[:, h * HEAD_DIM:(h + 1) * HEAD_DIM],
                     kvgw[:, h * HEAD_DIM:(h + 1) * HEAD_DIM],
                     preferred_element_type=jnp.float32)
             for h in range(HEADS)], axis=1)
        out_ref[w * ws:(w + 1) * ws, :] = num

    attn = (out_ref[...] * SCALE) / (jnp.abs(den * SCALE) + 1e-4)
    attn = _permute_rows(attn, _INV, _GW)
    y = jnp.dot(attn, wp_ref[...],
                preferred_element_type=jnp.float32) + bp_ref[...]
    y_ref[...] = _ln(y, gp_ref[...], bpn_ref[...])


@jax.jit
def kernel(x, W_qkv, gq, bq, gk, bk, gv, bv, Wp, bp, gp, bpn):
    T, B, Lt, Lh, Lw, C = x.shape
    wt, wh, ww = NWIN
    ws = (Lt // wt) * (Lh // wh) * (Lw // ww)
    TB = T * B
    n_rows = W * ws

    x2 = x.reshape(TB * n_rows, C)

    routing = pl.pallas_call(
        functools.partial(_routing_kernel, tb=TB, b_count=B, ws=ws),
        out_shape=(jax.ShapeDtypeStruct((B, W, TOPK_WIN), jnp.int32),
                   jax.ShapeDtypeStruct((B, W, W), jnp.float32)),
        in_specs=[pl.BlockSpec((TB * n_rows, C), lambda: (0, 0))],
        out_specs=(pl.BlockSpec((B, W, TOPK_WIN), lambda: (0, 0, 0)),
                   pl.BlockSpec((B, W, W), lambda: (0, 0, 0))),
            )
    idx, R = routing(x2)

    v1 = lambda a: a.reshape(1, C)
    full2 = lambda shape: pl.BlockSpec(shape, lambda i, idx_ref: (0, 0))

    main = pl.pallas_call(
        functools.partial(_main_kernel, ws=ws, b_count=B),
        grid_spec=pltpu.PrefetchScalarGridSpec(
            num_scalar_prefetch=1,
            grid=(TB,),
            in_specs=[
                pl.BlockSpec((n_rows, C), lambda i, idx_ref: (i, 0)),
                full2((C, 3 * C)),
                full2((1, C)), full2((1, C)),
                full2((1, C)), full2((1, C)),
                full2((1, C)), full2((1, C)),
                pl.BlockSpec((1, W, W),
                             lambda i, idx_ref: (i % B, 0, 0)),
                full2((C, C)),
                full2((1, C)), full2((1, C)), full2((1, C)),
            ],
            out_specs=pl.BlockSpec((n_rows, C), lambda i, idx_ref: (i, 0)),
            scratch_shapes=[
                pltpu.VMEM((W * HEAD_DIM, C), jnp.bfloat16),
                pltpu.VMEM((n_rows, C), jnp.float32),
            ],
        ),
        out_shape=jax.ShapeDtypeStruct((TB * n_rows, C), jnp.float32),
        compiler_params=pltpu.CompilerParams(
            dimension_semantics=("parallel",)),
            )
    y2 = main(idx, x2, W_qkv, v1(gq), v1(bq), v1(gk), v1(bk), v1(gv), v1(bv),
              R, Wp, v1(bp), v1(gp), v1(bpn))

    return y2.reshape(T, B, Lt, Lh, Lw, C)


# speedup vs baseline: 8.0001x; 1.0016x over previous
import functools

import jax
import jax.numpy as jnp
from jax.experimental import pallas as pl
from jax.experimental.pallas import tpu as pltpu

DIM = 256
HEADS = 8
HEAD_DIM = DIM // HEADS
NWIN = (2, 4, 4)
TOPK_WIN = 4
VTH = 0.5
TAU = 2.0
SCALE = HEAD_DIM ** (-0.5)

W = NWIN[0] * NWIN[1] * NWIN[2]

_LT, _LH, _LW = 8, 16, 16
_GT, _GH, _GW = _LT // NWIN[0], _LH // NWIN[1], _LW // NWIN[2]


def _fwd_piece_starts():
    starts = []
    for wt in range(NWIN[0]):
        for wh in range(NWIN[1]):
            for ww in range(NWIN[2]):
                for it in range(_GT):
                    for ih in range(_GH):
                        starts.append(((wt * _GT + it) * _LH
                                       + (wh * _GH + ih)) * _LW + ww * _GW)
    return starts


def _inv_piece_starts():
    ws = _GT * _GH * _GW
    starts = []
    for lt in range(_LT):
        for lh in range(_LH):
            for ww in range(NWIN[2]):
                w = (lt // _GT) * NWIN[1] * NWIN[2] + (lh // _GH) * NWIN[2] + ww
                starts.append(w * ws + (lt % _GT) * _GH * _GW
                              + (lh % _GH) * _GW)
    return starts


_FWD = _fwd_piece_starts()
_INV = _inv_piece_starts()


def _permute_rows(val, starts, run):
    return jnp.concatenate([val[s:s + run, :] for s in starts], axis=0)


def _win_id_iota(shape, dim):
    s = jax.lax.broadcasted_iota(jnp.int32, shape, dim)
    lt = s // (_LH * _LW)
    lh = (s // _LW) % _LH
    lw = s % _LW
    return ((lt // _GT) * NWIN[1] * NWIN[2]
            + (lh // _GH) * NWIN[2] + (lw // _GW))


def _ln(xv, g, b, eps=1e-5):
    m = jnp.mean(xv, axis=-1, keepdims=True)
    c = xv - m
    v = jnp.mean(c * c, axis=-1, keepdims=True)
    return c * jax.lax.rsqrt(v + eps) * g + b


def _spike_ln(xv, g, b, dtype, eps=1e-5):
    m = jnp.mean(xv, axis=-1, keepdims=True)
    c = xv - m
    v = jnp.mean(c * c, axis=-1, keepdims=True)
    thr = (VTH * TAU - b) * jnp.sqrt(v + eps)
    return jnp.where(c * g >= thr, 1.0, 0.0).astype(dtype)


def _routing_kernel(x_ref, idx_ref, r_ref, *, tb, b_count, ws):
    n_rows = W * ws
    t_count = tb // b_count
    sums = [None] * b_count
    for i in range(tb):
        chunk = x_ref[i * n_rows:(i + 1) * n_rows, :]
        b = i % b_count
        sums[b] = chunk if sums[b] is None else sums[b] + chunk
    for b in range(b_count):
        sums[b] = _permute_rows(sums[b], _FWD, _GW)
    fold = ws
    while fold > 8:
        fold //= 2
        for b in range(b_count):
            a = sums[b]
            sums[b] = jnp.concatenate(
                [a[w * 2 * fold:w * 2 * fold + fold, :]
                 + a[w * 2 * fold + fold:(w + 1) * 2 * fold, :]
                 for w in range(W)], axis=0)
    srow = jax.lax.broadcasted_iota(jnp.int32, (W, W * fold), 1)
    wrow = jax.lax.broadcasted_iota(jnp.int32, (W, W * fold), 0)
    S = jnp.where(srow // fold == wrow, 1.0, 0.0).astype(jnp.float32)
    colidx = jax.lax.broadcasted_iota(jnp.int32, (W, W), 1)
    for b in range(b_count):
        region = jnp.dot(S, sums[b], preferred_element_type=jnp.float32,
                         precision=jax.lax.Precision.HIGHEST)
        region = region * (1.0 / (t_count * ws))
        region16 = region.astype(jnp.bfloat16)
        scores = jax.lax.dot_general(
            region16, region16, (((1,), (1,)), ((), ())),
            preferred_element_type=jnp.float32) * SCALE
        masked = scores
        Rb = jnp.zeros((W, W), jnp.float32)
        for j in range(TOPK_WIN):
            mval = jnp.max(masked, axis=1, keepdims=True)
            eq = masked == mval
            cand = jnp.where(eq, colidx, jnp.int32(10 ** 9))
            cmin = jnp.min(cand, axis=1, keepdims=True)
            onehot = colidx == cmin
            Rb = Rb + onehot.astype(jnp.float32)
            masked = jnp.where(onehot, -jnp.inf, masked)
            idx_ref[b, :, j:j + 1] = cmin
        r_ref[b] = Rb


def _main_kernel(idx_sref, x_ref, wqkv_ref, gq_ref, bq_ref, gk_ref, bk_ref,
                 gv_ref, bv_ref, r_ref, wp_ref, bp_ref, gp_ref, bpn_ref,
                 y_ref, kvb_ref, out_ref, *, ws, b_count):
    b = pl.program_id(0) % b_count
    xv = _permute_rows(x_ref[...], _FWD, _GW)
    qkv = jnp.dot(xv, wqkv_ref[...],
                  preferred_element_type=jnp.float32)
    bf = jnp.bfloat16
    q16 = _spike_ln(qkv[:, 0:DIM], gq_ref[...], bq_ref[...], bf)
    k = _spike_ln(qkv[:, DIM:2 * DIM], gk_ref[...], bk_ref[...], bf)
    v = _spike_ln(qkv[:, 2 * DIM:3 * DIM], gv_ref[...], bv_ref[...], bf)

    n_rows = W * ws
    srow = jax.lax.broadcasted_iota(jnp.int32, (W, n_rows), 1)
    wrow = jax.lax.broadcasted_iota(jnp.int32, (W, n_rows), 0)
    S = jnp.where(srow // ws == wrow, 1.0, 0.0).astype(bf)
    ks = jnp.dot(S, k, preferred_element_type=jnp.float32)

    for w in range(W):
        kw = k[w * ws:(w + 1) * ws, :]
        vw = v[w * ws:(w + 1) * ws, :]
        kv_full = jax.lax.dot_general(
            kw, vw, (((0,), (0,)), ((), ())),
            preferred_element_type=jnp.float32)
        br = jnp.concatenate(
            [kv_full[h * HEAD_DIM:(h + 1) * HEAD_DIM,
                     h * HEAD_DIM:(h + 1) * HEAD_DIM]
             for h in range(HEADS)], axis=1)
        kvb_ref[w * HEAD_DIM:(w + 1) * HEAD_DIM, :] = br.astype(jnp.bfloat16)

    R = r_ref[0]
    ksg = jnp.dot(R, ks, preferred_element_type=jnp.float32)
    rrow = jax.lax.broadcasted_iota(jnp.int32, (n_rows, W), 0)
    rcol = jax.lax.broadcasted_iota(jnp.int32, (n_rows, W), 1)
    Urows = jnp.where(rrow // ws == rcol, 1.0, 0.0).astype(bf)
    ksg_exp = jnp.dot(Urows, ksg.astype(bf),
                      preferred_element_type=jnp.float32).astype(bf)
    ci = jax.lax.broadcasted_iota(jnp.int32, (DIM, HEADS), 0)
    hi = jax.lax.broadcasted_iota(jnp.int32, (DIM, HEADS), 1)
    Esum = jnp.where(ci // HEAD_DIM == hi, 1.0, 0.0).astype(jnp.float32)
    den8 = jnp.dot(q16 * ksg_exp, Esum.astype(bf),
                   preferred_element_type=jnp.float32)
    den = jnp.dot(den8, jnp.transpose(Esum),
                  preferred_element_type=jnp.float32)

    nums = []
    for w in range(W):
        j0 = idx_sref[b, w, 0]
        kvgw = kvb_ref[pl.ds(j0 * HEAD_DIM, HEAD_DIM), :]
        for j in range(1, TOPK_WIN):
            jj = idx_sref[b, w, j]
            kvgw = kvgw + kvb_ref[pl.ds(jj * HEAD_DIM, HEAD_DIM), :]
        qw = q16[w * ws:(w + 1) * ws, :]
        num = jnp.concatenate(
            [jnp.dot(qw[:, h * HEAD_DIM:(h + 1) * HEAD_DIM],
                     kvgw[:, h * HEAD_DIM:(h + 1) * HEAD_DIM],
                     preferred_element_type=jnp.float32)
             for h in range(HEADS)], axis=1)
        nums.append(num)

    out_all = jnp.concatenate(nums, axis=0)
    attn = (out_all * SCALE) / (jnp.abs(den * SCALE) + 1e-4)
    attn = _permute_rows(attn, _INV, _GW)
    y = jnp.dot(attn, wp_ref[...],
                preferred_element_type=jnp.float32) + bp_ref[...]
    y_ref[...] = _ln(y, gp_ref[...], bpn_ref[...])


@jax.jit
def kernel(x, W_qkv, gq, bq, gk, bk, gv, bv, Wp, bp, gp, bpn):
    T, B, Lt, Lh, Lw, C = x.shape
    wt, wh, ww = NWIN
    ws = (Lt // wt) * (Lh // wh) * (Lw // ww)
    TB = T * B
    n_rows = W * ws

    x2 = x.reshape(TB * n_rows, C)

    routing = pl.pallas_call(
        functools.partial(_routing_kernel, tb=TB, b_count=B, ws=ws),
        out_shape=(jax.ShapeDtypeStruct((B, W, TOPK_WIN), jnp.int32),
                   jax.ShapeDtypeStruct((B, W, W), jnp.float32)),
        in_specs=[pl.BlockSpec((TB * n_rows, C), lambda: (0, 0))],
        out_specs=(pl.BlockSpec((B, W, TOPK_WIN), lambda: (0, 0, 0)),
                   pl.BlockSpec((B, W, W), lambda: (0, 0, 0))),
            )
    idx, R = routing(x2)

    v1 = lambda a: a.reshape(1, C)
    full2 = lambda shape: pl.BlockSpec(shape, lambda i, idx_ref: (0, 0))

    main = pl.pallas_call(
        functools.partial(_main_kernel, ws=ws, b_count=B),
        grid_spec=pltpu.PrefetchScalarGridSpec(
            num_scalar_prefetch=1,
            grid=(TB,),
            in_specs=[
                pl.BlockSpec((n_rows, C), lambda i, idx_ref: (i, 0)),
                full2((C, 3 * C)),
                full2((1, C)), full2((1, C)),
                full2((1, C)), full2((1, C)),
                full2((1, C)), full2((1, C)),
                pl.BlockSpec((1, W, W),
                             lambda i, idx_ref: (i % B, 0, 0)),
                full2((C, C)),
                full2((1, C)), full2((1, C)), full2((1, C)),
            ],
            out_specs=pl.BlockSpec((n_rows, C), lambda i, idx_ref: (i, 0)),
            scratch_shapes=[
                pltpu.VMEM((W * HEAD_DIM, C), jnp.bfloat16),
                pltpu.VMEM((n_rows, C), jnp.float32),
            ],
        ),
        out_shape=jax.ShapeDtypeStruct((TB * n_rows, C), jnp.float32),
        compiler_params=pltpu.CompilerParams(
            dimension_semantics=("parallel",)),
            )
    y2 = main(idx, x2, W_qkv, v1(gq), v1(bq), v1(gk), v1(bk), v1(gv), v1(bv),
              R, Wp, v1(bp), v1(gp), v1(bpn))

    return y2.reshape(T, B, Lt, Lh, Lw, C)
